# Initial kernel scaffold; baseline (speedup 1.0000x reference)
import jax
import jax.numpy as jnp
from jax.experimental import pallas as pl


def kernel(x, edge_index, edge_attr, W_z, b_z, W_r, b_r, W_h, b_h, Lz_w, Lz_b, Lr_w, Lr_b, Lh_w, Lh_b, att, lin_w, lin_b):
    raise NotImplementedError("write your pallas kernel here")


# speedup vs baseline: 28.0498x; 28.0498x over previous
import jax
import jax.numpy as jnp
from jax.experimental import pallas as pl

OUT = 32
PERIODS = 12
NB = 2000


def _gru_body(s_ref, wb_ref, lb_ref, lwb_ref, o_ref):
    S = s_ref[...]
    WB = wb_ref[...]
    LB = lb_ref[...]
    LWB = lwb_ref[...]
    f32 = jnp.float32

    H = jnp.zeros((S.shape[0], OUT), f32)
    acc = jnp.zeros((S.shape[0], OUT), f32)
    for p in range(PERIODS):
        s0 = S[:, 2 * p:2 * p + 1]
        s1 = S[:, 2 * p + 1:2 * p + 2]
        az = s0 * WB[0:1, :] + s1 * WB[1:2, :] + WB[6:7, :]
        ar = s0 * WB[2:3, :] + s1 * WB[3:4, :] + WB[7:8, :]
        ah = s0 * WB[4:5, :] + s1 * WB[5:6, :] + WB[8:9, :]
        Z = jax.nn.sigmoid(az + jnp.dot(H, LB[0:32, :], preferred_element_type=f32))
        R = jax.nn.sigmoid(ar + jnp.dot(H, LB[32:64, :], preferred_element_type=f32))
        Ht = jnp.tanh(ah + jnp.dot(H * R, LB[64:96, :], preferred_element_type=f32))
        H = Z * H + (1.0 - Z) * Ht
        acc = acc + WB[9:10, p:p + 1] * H
    h = jax.nn.relu(acc)
    o_ref[...] = jnp.dot(h, LWB[0:32, :], preferred_element_type=f32) + LWB[32:33, :]


def _gru_pallas(S, WB, LB, LWB, n):
    return pl.pallas_call(
        _gru_body,
        grid=(n // NB,),
        in_specs=[
            pl.BlockSpec((NB, 2 * PERIODS), lambda i: (i, 0)),
            pl.BlockSpec((16, OUT), lambda i: (0, 0)),
            pl.BlockSpec((96, OUT), lambda i: (0, 0)),
            pl.BlockSpec((40, PERIODS), lambda i: (0, 0)),
        ],
        out_specs=pl.BlockSpec((NB, PERIODS), lambda i: (i, 0)),
        out_shape=jax.ShapeDtypeStruct((n, PERIODS), jnp.float32),
    )(S, WB, LB, LWB)


def kernel(x, edge_index, edge_attr, W_z, b_z, W_r, b_r, W_h, b_h,
           Lz_w, Lz_b, Lr_w, Lr_b, Lh_w, Lh_b, att, lin_w, lin_b):
    n = x.shape[0]
    src = edge_index[0]
    dst = edge_index[1]

    deg = jnp.zeros((n,), jnp.float32).at[dst].add(edge_attr) + 1.0
    dinv = jax.lax.rsqrt(deg)
    X24 = x.transpose(0, 2, 1).reshape(n, 2 * PERIODS)
    Y = dinv[:, None] * X24
    T = jnp.zeros((n, 2 * PERIODS), jnp.float32).at[dst].add(
        edge_attr[:, None] * Y[src])
    S = dinv[:, None] * T + (dinv * dinv)[:, None] * X24

    probs = jax.nn.softmax(att)
    Wcz = W_z @ Lz_w[:OUT]; bcz = b_z @ Lz_w[:OUT] + Lz_b
    Wcr = W_r @ Lr_w[:OUT]; bcr = b_r @ Lr_w[:OUT] + Lr_b
    Wch = W_h @ Lh_w[:OUT]; bch = b_h @ Lh_w[:OUT] + Lh_b
    probs_row = jnp.zeros((OUT,), jnp.float32).at[:PERIODS].set(probs)
    WB = jnp.zeros((16, OUT), jnp.float32)
    WB = WB.at[0:2].set(Wcz).at[2:4].set(Wcr).at[4:6].set(Wch)
    WB = WB.at[6].set(bcz).at[7].set(bcr).at[8].set(bch).at[9].set(probs_row)
    LB = jnp.concatenate([Lz_w[OUT:], Lr_w[OUT:], Lh_w[OUT:]], axis=0)
    LWB = jnp.zeros((40, PERIODS), jnp.float32)
    LWB = LWB.at[0:OUT].set(lin_w).at[OUT].set(lin_b)

    return _gru_pallas(S, WB, LB, LWB, n)
